# Initial kernel scaffold; baseline (speedup 1.0000x reference)
import functools

import jax
import jax.numpy as jnp
from jax import lax
from jax.experimental import pallas as pl
from jax.experimental.pallas import tpu as pltpu
from jax.experimental.pallas import tpu_sc as plsc

N = 10000
NPAD = 10240
D = 128
E = 320000
G = 64

NT = 16
K = 128
EPT = E // NT
CH = -(-EPT // K)
EPTP = CH * K
RPT = NPAD // NT
DW = 16

BLK = 256
NB = NPAD // BLK

_mesh = plsc.VectorSubcoreMesh(core_axis_name="c", subcore_axis_name="s")
_F32 = jnp.float32
_HI = jax.lax.Precision.HIGHEST


@functools.partial(
    pl.kernel,
    out_type=(jax.ShapeDtypeStruct((NPAD, DW), _F32),
              jax.ShapeDtypeStruct((NPAD, DW), _F32)),
    scratch_types=[
        pltpu.VMEM((K,), jnp.int32),
        pltpu.VMEM((K, DW), _F32),
        pltpu.VMEM_SHARED((NPAD, DW), _F32),
    ],
    mesh=_mesh,
)
def _sc_degrees(rowp_h, colp_h, dout_h, din_h, idx_v, ones_v, acc):
    c = lax.axis_index("c")
    s = lax.axis_index("s")

    def direction(src_h, out_h):
        def zfill(j, _):
            ones_v[j, :] = jnp.zeros((DW,), _F32)
            return 0
        lax.fori_loop(0, K, zfill, 0)

        def zacc(i, _):
            pltpu.sync_copy(ones_v, acc.at[pl.ds(s * RPT + i * K, K)])
            return 0
        lax.fori_loop(0, RPT // K, zacc, 0)

        def ofill(j, _):
            ones_v[j, :] = jnp.ones((DW,), _F32)
            return 0
        lax.fori_loop(0, K, ofill, 0)

        plsc.subcore_barrier()

        def step(i, _):
            pltpu.sync_copy(src_h.at[s, pl.ds(i * K, K)], idx_v)
            pltpu.sync_copy(ones_v, acc.at[idx_v], add=True)
            return 0
        lax.fori_loop(0, CH, step, 0)

        plsc.subcore_barrier()
        pltpu.sync_copy(acc.at[pl.ds(s * RPT, RPT)], out_h.at[pl.ds(s * RPT, RPT)])

    @pl.when(c == 0)
    def _():
        direction(rowp_h, dout_h)

    @pl.when(c == 1)
    def _():
        direction(colp_h, din_h)


@functools.partial(
    pl.kernel,
    out_type=(jax.ShapeDtypeStruct((NPAD, D), _F32),
              jax.ShapeDtypeStruct((NPAD, D), _F32)),
    scratch_types=[
        pltpu.VMEM((K,), jnp.int32),
        pltpu.VMEM((K,), jnp.int32),
        pltpu.VMEM((K, D), _F32),
        pltpu.VMEM_SHARED((NPAD, D), _F32),
        pltpu.SemaphoreType.DMA,
    ],
    mesh=_mesh,
)
def _sc_spmm(u_h, v_h, rowp_h, colp_h, su_h, stv_h, gi, si, rows, acc, sem):
    c = lax.axis_index("c")
    s = lax.axis_index("s")

    def direction(src_h, gidx_h, sidx_h, out_h):
        def zfill(i, _):
            rows[i // 8, pl.ds((i % 8) * 16, 16)] = jnp.zeros((16,), _F32)
            return 0
        lax.fori_loop(0, K * 8, zfill, 0)

        def zacc(i, _):
            pltpu.sync_copy(rows, acc.at[pl.ds(s * RPT + i * K, K)])
            return 0
        lax.fori_loop(0, RPT // K, zacc, 0)

        plsc.subcore_barrier()

        def step(i, _):
            pltpu.sync_copy(gidx_h.at[s, pl.ds(i * K, K)], gi)
            pltpu.sync_copy(sidx_h.at[s, pl.ds(i * K, K)], si)
            pltpu.async_copy(src_h.at[gi], rows, sem).wait()
            pltpu.sync_copy(rows, acc.at[si], add=True)
            return 0
        lax.fori_loop(0, CH, step, 0)

        plsc.subcore_barrier()
        pltpu.sync_copy(acc.at[pl.ds(s * RPT, RPT)], out_h.at[pl.ds(s * RPT, RPT)])

    @pl.when(c == 0)
    def _():
        direction(u_h, colp_h, rowp_h, su_h)

    @pl.when(c == 1)
    def _():
        direction(v_h, rowp_h, colp_h, stv_h)


def _prep_body(x_ref, do_ref, di_ref, u_ref, v_ref, io_ref, ii_ref):
    d_o = do_ref[:, 0:1]
    d_i = di_ref[:, 0:1]
    io = jnp.where(d_o > 0, lax.rsqrt(d_o), 0.0)
    ii = jnp.where(d_i > 0, lax.rsqrt(d_i), 0.0)
    xb = x_ref[...]
    u_ref[...] = ii * xb
    v_ref[...] = io * xb
    io_ref[...] = jnp.broadcast_to(io, (BLK, DW))
    ii_ref[...] = jnp.broadcast_to(ii, (BLK, DW))


def _tc_prep(x_p, dout, din):
    bs_d = pl.BlockSpec((BLK, D), lambda i: (i, 0))
    bs_w = pl.BlockSpec((BLK, DW), lambda i: (i, 0))
    return pl.pallas_call(
        _prep_body,
        grid=(NB,),
        in_specs=[bs_d, bs_w, bs_w],
        out_specs=[bs_d, bs_d, bs_w, bs_w],
        out_shape=[jax.ShapeDtypeStruct((NPAD, D), _F32),
                   jax.ShapeDtypeStruct((NPAD, D), _F32),
                   jax.ShapeDtypeStruct((NPAD, DW), _F32),
                   jax.ShapeDtypeStruct((NPAD, DW), _F32)],
    )(x_p, dout, din)


def _combine_h(su_ref, st_ref, io_ref, ii_ref, ws_ref, bs_ref, wd_ref, bd_ref):
    io = io_ref[:, 0:1]
    ii = ii_ref[:, 0:1]
    a = io * su_ref[...]
    b = ii * st_ref[...]
    h = 0.5 * (jnp.dot(a, ws_ref[...], precision=_HI, preferred_element_type=_F32)
               + bs_ref[0:1, :])
    h += 0.5 * (jnp.dot(b, wd_ref[...], precision=_HI, preferred_element_type=_F32)
                + bd_ref[0:1, :])
    return jnp.maximum(h, 0.0)


def _combine_body(su_ref, st_ref, io_ref, ii_ref, ws_ref, bs_ref, wd_ref, bd_ref,
                  u_ref, v_ref):
    h = _combine_h(su_ref, st_ref, io_ref, ii_ref, ws_ref, bs_ref, wd_ref, bd_ref)
    u_ref[...] = ii_ref[:, 0:1] * h
    v_ref[...] = io_ref[:, 0:1] * h


def _tc_combine(su, stv, io16, ii16, Ws, bsb, Wd, bdb):
    bs_d = pl.BlockSpec((BLK, D), lambda i: (i, 0))
    bs_w = pl.BlockSpec((BLK, DW), lambda i: (i, 0))
    bs_m = pl.BlockSpec((D, D), lambda i: (0, 0))
    bs_b = pl.BlockSpec((8, D), lambda i: (0, 0))
    return pl.pallas_call(
        _combine_body,
        grid=(NB,),
        in_specs=[bs_d, bs_d, bs_w, bs_w, bs_m, bs_b, bs_m, bs_b],
        out_specs=[bs_d, bs_d],
        out_shape=[jax.ShapeDtypeStruct((NPAD, D), _F32),
                   jax.ShapeDtypeStruct((NPAD, D), _F32)],
    )(su, stv, io16, ii16, Ws, bsb, Wd, bdb)


def _final_body(su_ref, st_ref, io_ref, ii_ref, ws_ref, bs_ref, wd_ref, bd_ref,
                batch_ref, wl1_ref, bl1_ref, wl2_ref, bl2_ref, out_ref, pool_scr):
    i = pl.program_id(0)

    @pl.when(i == 0)
    def _():
        pool_scr[...] = jnp.full((G, D), -jnp.inf, _F32)

    h = _combine_h(su_ref, st_ref, io_ref, ii_ref, ws_ref, bs_ref, wd_ref, bd_ref)
    b = batch_ref[0, 0, :]

    acc = pool_scr[...]
    for g in range(G):
        m = (b == g)[:, None]
        red = jnp.max(jnp.where(m, h, -jnp.inf), axis=0, keepdims=True)
        acc = acc.at[g:g + 1, :].set(jnp.maximum(acc[g:g + 1, :], red))
    pool_scr[...] = acc

    @pl.when(i == NB - 1)
    def _():
        p = pool_scr[...]
        o1 = jnp.maximum(
            jnp.dot(p, wl1_ref[...], precision=_HI, preferred_element_type=_F32)
            + bl1_ref[0:1, :], 0.0)
        out_ref[...] = (jnp.dot(o1, wl2_ref[...], precision=_HI,
                                preferred_element_type=_F32) + bl2_ref[0:1, :])


def _tc_final(su, stv, io16, ii16, Ws, bsb, Wd, bdb, batch3, Wl1p, bl1b, Wl2p, bl2b):
    bs_d = pl.BlockSpec((BLK, D), lambda i: (i, 0))
    bs_w = pl.BlockSpec((BLK, DW), lambda i: (i, 0))
    bs_m = pl.BlockSpec((D, D), lambda i: (0, 0))
    bs_b = pl.BlockSpec((8, D), lambda i: (0, 0))
    bs_i = pl.BlockSpec((1, 1, BLK), lambda i: (i, 0, 0))
    return pl.pallas_call(
        _final_body,
        grid=(NB,),
        in_specs=[bs_d, bs_d, bs_w, bs_w, bs_m, bs_b, bs_m, bs_b,
                  bs_i, bs_m, bs_b, bs_m, bs_b],
        out_specs=pl.BlockSpec((G, D), lambda i: (0, 0)),
        out_shape=jax.ShapeDtypeStruct((G, D), _F32),
        scratch_shapes=[pltpu.VMEM((G, D), _F32)],
    )(su, stv, io16, ii16, Ws, bsb, Wd, bdb, batch3, Wl1p, bl1b, Wl2p, bl2b)


def kernel(x, edge_index, batch, W1s, b1s, W1d, b1d, W2s, b2s, W2d, b2d,
           W3s, b3s, W3d, b3d, Wl1, bl1, Wl2, bl2):
    row, col = edge_index[0], edge_index[1]

    pad = jnp.full((NT, EPTP - EPT), NPAD - 1, jnp.int32)
    rowp = jnp.concatenate([row.reshape(NT, EPT), pad], axis=1)
    colp = jnp.concatenate([col.reshape(NT, EPT), pad], axis=1)

    x_p = jnp.pad(x, ((0, NPAD - N), (0, 0)))
    batch3 = jnp.pad(batch, (0, NPAD - N), constant_values=G).reshape(NB, 1, BLK)

    bcast8 = lambda vec: jnp.broadcast_to(
        jnp.pad(vec, (0, D - vec.shape[0]))[None, :], (8, D))
    b1sb, b1db = bcast8(b1s), bcast8(b1d)
    b2sb, b2db = bcast8(b2s), bcast8(b2d)
    b3sb, b3db = bcast8(b3s), bcast8(b3d)
    Wl1p = jnp.zeros((D, D), _F32).at[:, :Wl1.shape[1]].set(Wl1)
    Wl2p = jnp.zeros((D, D), _F32).at[:Wl2.shape[0], :Wl2.shape[1]].set(Wl2)
    bl1b, bl2b = bcast8(bl1), bcast8(bl2)

    dout, din = _sc_degrees(rowp, colp)
    u, v, io16, ii16 = _tc_prep(x_p, dout, din)

    su, stv = _sc_spmm(u, v, rowp, colp)
    u, v = _tc_combine(su, stv, io16, ii16, W1s, b1sb, W1d, b1db)

    su, stv = _sc_spmm(u, v, rowp, colp)
    u, v = _tc_combine(su, stv, io16, ii16, W2s, b2sb, W2d, b2db)

    su, stv = _sc_spmm(u, v, rowp, colp)
    out128 = _tc_final(su, stv, io16, ii16, W3s, b3sb, W3d, b3db,
                       batch3, Wl1p, bl1b, Wl2p, bl2b)

    return out128[:, :1]


# speedup vs baseline: 7.6922x; 7.6922x over previous
import functools

import jax
import jax.numpy as jnp
from jax import lax
from jax.experimental import pallas as pl
from jax.experimental.pallas import tpu as pltpu
from jax.experimental.pallas import tpu_sc as plsc

N = 10000
NPAD = 10240
D = 128
E = 320000
G = 64

NT = 16
K = 128
EPT = E // NT
CH = -(-EPT // K)
EPTP = CH * K
RPT = NPAD // NT
DW = 16

BLK = 256
NB = NPAD // BLK

_mesh = plsc.VectorSubcoreMesh(core_axis_name="c", subcore_axis_name="s")
_F32 = jnp.float32
_HI = jax.lax.Precision.HIGHEST


@functools.partial(
    pl.kernel,
    out_type=(jax.ShapeDtypeStruct((NPAD, DW), _F32),
              jax.ShapeDtypeStruct((NPAD, DW), _F32)),
    scratch_types=[
        pltpu.VMEM((K,), jnp.int32),
        pltpu.VMEM((K, DW), _F32),
        pltpu.VMEM_SHARED((NPAD, DW), _F32),
    ],
    mesh=_mesh,
)
def _sc_degrees(rowp_h, colp_h, dout_h, din_h, idx_v, ones_v, acc):
    c = lax.axis_index("c")
    s = lax.axis_index("s")

    def direction(src_h, out_h):
        def zfill(j, _):
            ones_v[j, :] = jnp.zeros((DW,), _F32)
            return 0
        lax.fori_loop(0, K, zfill, 0)

        def zacc(i, _):
            pltpu.sync_copy(ones_v, acc.at[pl.ds(s * RPT + i * K, K)])
            return 0
        lax.fori_loop(0, RPT // K, zacc, 0)

        def ofill(j, _):
            ones_v[j, :] = jnp.ones((DW,), _F32)
            return 0
        lax.fori_loop(0, K, ofill, 0)

        plsc.subcore_barrier()

        def step(i, _):
            pltpu.sync_copy(src_h.at[s, pl.ds(i * K, K)], idx_v)
            pltpu.sync_copy(ones_v, acc.at[idx_v], add=True)
            return 0
        lax.fori_loop(0, CH, step, 0)

        plsc.subcore_barrier()
        pltpu.sync_copy(acc.at[pl.ds(s * RPT, RPT)], out_h.at[pl.ds(s * RPT, RPT)])

    @pl.when(c == 0)
    def _():
        direction(rowp_h, dout_h)

    @pl.when(c == 1)
    def _():
        direction(colp_h, din_h)


@functools.partial(
    pl.kernel,
    out_type=(jax.ShapeDtypeStruct((NPAD, D), _F32),
              jax.ShapeDtypeStruct((NPAD, D), _F32)),
    scratch_types=[
        pltpu.VMEM((K,), jnp.int32),
        pltpu.VMEM((K,), jnp.int32),
        pltpu.VMEM((K, D), _F32),
        pltpu.VMEM_SHARED((NPAD, D), _F32),
        pltpu.SemaphoreType.DMA,
    ],
    mesh=_mesh,
)
def _sc_spmm(u_h, v_h, rowp_h, colp_h, su_h, stv_h, gi, si, rows, acc, sem):
    c = lax.axis_index("c")
    s = lax.axis_index("s")

    def direction(src_h, gidx_h, sidx_h, out_h):
        def zfill(i, _):
            rows[i // 8, pl.ds((i % 8) * 16, 16)] = jnp.zeros((16,), _F32)
            return 0
        lax.fori_loop(0, K * 8, zfill, 0)

        def zacc(i, _):
            pltpu.sync_copy(rows, acc.at[pl.ds(s * RPT + i * K, K)])
            return 0
        lax.fori_loop(0, RPT // K, zacc, 0)

        plsc.subcore_barrier()

        def step(i, _):
            pltpu.sync_copy(gidx_h.at[s, pl.ds(i * K, K)], gi)
            pltpu.sync_copy(sidx_h.at[s, pl.ds(i * K, K)], si)
            pltpu.async_copy(src_h.at[gi], rows, sem).wait()
            pltpu.sync_copy(rows, acc.at[si], add=True)
            return 0
        lax.fori_loop(0, CH, step, 0)

        plsc.subcore_barrier()
        pltpu.sync_copy(acc.at[pl.ds(s * RPT, RPT)], out_h.at[pl.ds(s * RPT, RPT)])

    @pl.when(c == 0)
    def _():
        direction(u_h, colp_h, rowp_h, su_h)

    @pl.when(c == 1)
    def _():
        direction(v_h, rowp_h, colp_h, stv_h)


def _prep_body(x_ref, do_ref, di_ref, u_ref, v_ref, io_ref, ii_ref):
    d_o = do_ref[:, 0:1]
    d_i = di_ref[:, 0:1]
    io = jnp.where(d_o > 0, lax.rsqrt(d_o), 0.0)
    ii = jnp.where(d_i > 0, lax.rsqrt(d_i), 0.0)
    xb = x_ref[...]
    u_ref[...] = ii * xb
    v_ref[...] = io * xb
    io_ref[...] = jnp.broadcast_to(io, (BLK, DW))
    ii_ref[...] = jnp.broadcast_to(ii, (BLK, DW))


def _tc_prep(x_p, dout, din):
    bs_d = pl.BlockSpec((BLK, D), lambda i: (i, 0))
    bs_w = pl.BlockSpec((BLK, DW), lambda i: (i, 0))
    return pl.pallas_call(
        _prep_body,
        grid=(NB,),
        in_specs=[bs_d, bs_w, bs_w],
        out_specs=[bs_d, bs_d, bs_w, bs_w],
        out_shape=[jax.ShapeDtypeStruct((NPAD, D), _F32),
                   jax.ShapeDtypeStruct((NPAD, D), _F32),
                   jax.ShapeDtypeStruct((NPAD, DW), _F32),
                   jax.ShapeDtypeStruct((NPAD, DW), _F32)],
    )(x_p, dout, din)


def _combine_h(su_ref, st_ref, io_ref, ii_ref, ws_ref, bs_ref, wd_ref, bd_ref):
    io = io_ref[:, 0:1]
    ii = ii_ref[:, 0:1]
    a = io * su_ref[...]
    b = ii * st_ref[...]
    h = 0.5 * (jnp.dot(a, ws_ref[...], precision=_HI, preferred_element_type=_F32)
               + bs_ref[0:1, :])
    h += 0.5 * (jnp.dot(b, wd_ref[...], precision=_HI, preferred_element_type=_F32)
                + bd_ref[0:1, :])
    return jnp.maximum(h, 0.0)


def _combine_body(su_ref, st_ref, io_ref, ii_ref, ws_ref, bs_ref, wd_ref, bd_ref,
                  u_ref, v_ref):
    h = _combine_h(su_ref, st_ref, io_ref, ii_ref, ws_ref, bs_ref, wd_ref, bd_ref)
    u_ref[...] = ii_ref[:, 0:1] * h
    v_ref[...] = io_ref[:, 0:1] * h


def _tc_combine(su, stv, io16, ii16, Ws, bsb, Wd, bdb):
    bs_d = pl.BlockSpec((BLK, D), lambda i: (i, 0))
    bs_w = pl.BlockSpec((BLK, DW), lambda i: (i, 0))
    bs_m = pl.BlockSpec((D, D), lambda i: (0, 0))
    bs_b = pl.BlockSpec((8, D), lambda i: (0, 0))
    return pl.pallas_call(
        _combine_body,
        grid=(NB,),
        in_specs=[bs_d, bs_d, bs_w, bs_w, bs_m, bs_b, bs_m, bs_b],
        out_specs=[bs_d, bs_d],
        out_shape=[jax.ShapeDtypeStruct((NPAD, D), _F32),
                   jax.ShapeDtypeStruct((NPAD, D), _F32)],
    )(su, stv, io16, ii16, Ws, bsb, Wd, bdb)


def _final_body(su_ref, st_ref, io_ref, ii_ref, ws_ref, bs_ref, wd_ref, bd_ref,
                batch_ref, wl1_ref, bl1_ref, wl2_ref, bl2_ref, out_ref, pool_scr):
    i = pl.program_id(0)

    @pl.when(i == 0)
    def _():
        pool_scr[...] = jnp.full((G, D), -jnp.inf, _F32)

    h = _combine_h(su_ref, st_ref, io_ref, ii_ref, ws_ref, bs_ref, wd_ref, bd_ref)
    b = batch_ref[:, 0:1]

    for g in range(G):
        m = (b == g)
        red = jnp.max(jnp.where(m, h, -jnp.inf), axis=0, keepdims=True)
        pool_scr[g:g + 1, :] = jnp.maximum(pool_scr[g:g + 1, :], red)

    @pl.when(i == NB - 1)
    def _():
        p = pool_scr[...]
        o1 = jnp.maximum(
            jnp.dot(p, wl1_ref[...], precision=_HI, preferred_element_type=_F32)
            + bl1_ref[0:1, :], 0.0)
        out_ref[...] = (jnp.dot(o1, wl2_ref[...], precision=_HI,
                                preferred_element_type=_F32) + bl2_ref[0:1, :])


def _tc_final(su, stv, io16, ii16, Ws, bsb, Wd, bdb, batch3, Wl1p, bl1b, Wl2p, bl2b):
    bs_d = pl.BlockSpec((BLK, D), lambda i: (i, 0))
    bs_w = pl.BlockSpec((BLK, DW), lambda i: (i, 0))
    bs_m = pl.BlockSpec((D, D), lambda i: (0, 0))
    bs_b = pl.BlockSpec((8, D), lambda i: (0, 0))
    bs_i = pl.BlockSpec((BLK, DW), lambda i: (i, 0))
    return pl.pallas_call(
        _final_body,
        grid=(NB,),
        in_specs=[bs_d, bs_d, bs_w, bs_w, bs_m, bs_b, bs_m, bs_b,
                  bs_i, bs_m, bs_b, bs_m, bs_b],
        out_specs=pl.BlockSpec((G, D), lambda i: (0, 0)),
        out_shape=jax.ShapeDtypeStruct((G, D), _F32),
        scratch_shapes=[pltpu.VMEM((G, D), _F32)],
    )(su, stv, io16, ii16, Ws, bsb, Wd, bdb, batch3, Wl1p, bl1b, Wl2p, bl2b)


def kernel(x, edge_index, batch, W1s, b1s, W1d, b1d, W2s, b2s, W2d, b2d,
           W3s, b3s, W3d, b3d, Wl1, bl1, Wl2, bl2):
    row, col = edge_index[0], edge_index[1]

    pad = jnp.full((NT, EPTP - EPT), NPAD - 1, jnp.int32)
    rowp = jnp.concatenate([row.reshape(NT, EPT), pad], axis=1)
    colp = jnp.concatenate([col.reshape(NT, EPT), pad], axis=1)

    x_p = jnp.pad(x, ((0, NPAD - N), (0, 0)))
    batch3 = jnp.broadcast_to(
        jnp.pad(batch, (0, NPAD - N), constant_values=G)[:, None], (NPAD, DW))

    bcast8 = lambda vec: jnp.broadcast_to(
        jnp.pad(vec, (0, D - vec.shape[0]))[None, :], (8, D))
    b1sb, b1db = bcast8(b1s), bcast8(b1d)
    b2sb, b2db = bcast8(b2s), bcast8(b2d)
    b3sb, b3db = bcast8(b3s), bcast8(b3d)
    Wl1p = jnp.zeros((D, D), _F32).at[:, :Wl1.shape[1]].set(Wl1)
    Wl2p = jnp.zeros((D, D), _F32).at[:Wl2.shape[0], :Wl2.shape[1]].set(Wl2)
    bl1b, bl2b = bcast8(bl1), bcast8(bl2)

    dout, din = _sc_degrees(rowp, colp)
    u, v, io16, ii16 = _tc_prep(x_p, dout, din)

    su, stv = _sc_spmm(u, v, rowp, colp)
    u, v = _tc_combine(su, stv, io16, ii16, W1s, b1sb, W1d, b1db)

    su, stv = _sc_spmm(u, v, rowp, colp)
    u, v = _tc_combine(su, stv, io16, ii16, W2s, b2sb, W2d, b2db)

    su, stv = _sc_spmm(u, v, rowp, colp)
    out128 = _tc_final(su, stv, io16, ii16, W3s, b3sb, W3d, b3db,
                       batch3, Wl1p, bl1b, Wl2p, bl2b)

    return out128[:, :1]


# speedup vs baseline: 10.2310x; 1.3300x over previous
import functools

import jax
import jax.numpy as jnp
from jax import lax
from jax.experimental import pallas as pl
from jax.experimental.pallas import tpu as pltpu
from jax.experimental.pallas import tpu_sc as plsc

N = 10000
NPAD = 10240
D = 128
E = 320000
G = 64

NT = 16
K = 128
EPT = E // NT
CH = 2 * (-(-EPT // (2 * K)))
EPTP = CH * K
RPT = NPAD // NT
DW = 16

BLK = 256
NB = NPAD // BLK

_mesh = plsc.VectorSubcoreMesh(core_axis_name="c", subcore_axis_name="s")
_F32 = jnp.float32
_HI = jax.lax.Precision.HIGHEST


@functools.partial(
    pl.kernel,
    out_type=(jax.ShapeDtypeStruct((NPAD, DW), _F32),
              jax.ShapeDtypeStruct((NPAD, DW), _F32)),
    scratch_types=[
        pltpu.VMEM((K,), jnp.int32),
        pltpu.VMEM((K, DW), _F32),
        pltpu.VMEM_SHARED((NPAD, DW), _F32),
    ],
    mesh=_mesh,
)
def _sc_degrees(rowp_h, colp_h, dout_h, din_h, idx_v, ones_v, acc):
    c = lax.axis_index("c")
    s = lax.axis_index("s")

    def direction(src_h, out_h):
        def zfill(j, _):
            ones_v[j, :] = jnp.zeros((DW,), _F32)
            return 0
        lax.fori_loop(0, K, zfill, 0)

        def zacc(i, _):
            pltpu.sync_copy(ones_v, acc.at[pl.ds(s * RPT + i * K, K)])
            return 0
        lax.fori_loop(0, RPT // K, zacc, 0)

        def ofill(j, _):
            ones_v[j, :] = jnp.ones((DW,), _F32)
            return 0
        lax.fori_loop(0, K, ofill, 0)

        plsc.subcore_barrier()

        def step(i, _):
            pltpu.sync_copy(src_h.at[s, i], idx_v)
            pltpu.sync_copy(ones_v, acc.at[idx_v], add=True)
            return 0
        lax.fori_loop(0, CH, step, 0)

        plsc.subcore_barrier()
        pltpu.sync_copy(acc.at[pl.ds(s * RPT, RPT)], out_h.at[pl.ds(s * RPT, RPT)])

    @pl.when(c == 0)
    def _():
        direction(rowp_h, dout_h)

    @pl.when(c == 1)
    def _():
        direction(colp_h, din_h)


@functools.partial(
    pl.kernel,
    out_type=(jax.ShapeDtypeStruct((NPAD, D), _F32),
              jax.ShapeDtypeStruct((NPAD, D), _F32)),
    scratch_types=[
        pltpu.VMEM((K,), jnp.int32),
        pltpu.VMEM((K,), jnp.int32),
        pltpu.VMEM((K,), jnp.int32),
        pltpu.VMEM((K,), jnp.int32),
        pltpu.VMEM((K, D), _F32),
        pltpu.VMEM((K, D), _F32),
        pltpu.VMEM_SHARED((NPAD, D), _F32),
        pltpu.SemaphoreType.DMA,
        pltpu.SemaphoreType.DMA,
        pltpu.SemaphoreType.DMA,
        pltpu.SemaphoreType.DMA,
    ],
    mesh=_mesh,
)
def _sc_spmm(u_h, v_h, rowp_h, colp_h, su_h, stv_h, g0, g1, s0, s1,
             rows0, rows1, acc, smg0, smg1, smi0, smi1):
    c = lax.axis_index("c")
    s = lax.axis_index("s")

    def direction(src_h, gidx_h, sidx_h, out_h):
        def widx(buf, sem):
            pltpu.make_async_copy(gidx_h.at[s, 0], buf, sem).wait()

        def zfill(i, _):
            for l in range(8):
                rows0[i, pl.ds(l * 16, 16)] = jnp.zeros((16,), _F32)
            return 0
        lax.fori_loop(0, K, zfill, 0)

        def zacc(i, _):
            pltpu.sync_copy(rows0, acc.at[pl.ds(s * RPT + i * K, K)])
            return 0
        lax.fori_loop(0, RPT // K, zacc, 0)

        plsc.subcore_barrier()

        pltpu.sync_copy(gidx_h.at[s, 0], g0)
        pltpu.sync_copy(sidx_h.at[s, 0], s0)
        pltpu.async_copy(src_h.at[g0], rows0, smg0)
        pltpu.async_copy(gidx_h.at[s, 1], g1, smi1)
        pltpu.async_copy(sidx_h.at[s, 1], s1, smi1)

        def step(j, _):
            i0 = 2 * j
            widx(g1, smi1)
            widx(s1, smi1)
            h1 = pltpu.async_copy(src_h.at[g1], rows1, smg1)
            pltpu.make_async_copy(src_h.at[pl.ds(0, K)], rows0, smg0).wait()
            pltpu.sync_copy(rows0, acc.at[s0], add=True)

            @pl.when(i0 + 2 < CH)
            def _():
                pltpu.async_copy(gidx_h.at[s, i0 + 2], g0, smi0)
                pltpu.async_copy(sidx_h.at[s, i0 + 2], s0, smi0)
                widx(g0, smi0)
                widx(s0, smi0)
                pltpu.async_copy(src_h.at[g0], rows0, smg0)

            h1.wait()
            pltpu.sync_copy(rows1, acc.at[s1], add=True)

            @pl.when(i0 + 3 < CH)
            def _():
                pltpu.async_copy(gidx_h.at[s, i0 + 3], g1, smi1)
                pltpu.async_copy(sidx_h.at[s, i0 + 3], s1, smi1)
            return 0
        lax.fori_loop(0, CH // 2, step, 0)

        plsc.subcore_barrier()
        pltpu.sync_copy(acc.at[pl.ds(s * RPT, RPT)], out_h.at[pl.ds(s * RPT, RPT)])

    @pl.when(c == 0)
    def _():
        direction(u_h, colp_h, rowp_h, su_h)

    @pl.when(c == 1)
    def _():
        direction(v_h, rowp_h, colp_h, stv_h)


def _prep_body(x_ref, do_ref, di_ref, u_ref, v_ref, io_ref, ii_ref):
    d_o = do_ref[:, 0:1]
    d_i = di_ref[:, 0:1]
    io = jnp.where(d_o > 0, lax.rsqrt(d_o), 0.0)
    ii = jnp.where(d_i > 0, lax.rsqrt(d_i), 0.0)
    xb = x_ref[...]
    u_ref[...] = ii * xb
    v_ref[...] = io * xb
    io_ref[...] = jnp.broadcast_to(io, (BLK, DW))
    ii_ref[...] = jnp.broadcast_to(ii, (BLK, DW))


def _tc_prep(x_p, dout, din):
    bs_d = pl.BlockSpec((BLK, D), lambda i: (i, 0))
    bs_w = pl.BlockSpec((BLK, DW), lambda i: (i, 0))
    return pl.pallas_call(
        _prep_body,
        grid=(NB,),
        in_specs=[bs_d, bs_w, bs_w],
        out_specs=[bs_d, bs_d, bs_w, bs_w],
        out_shape=[jax.ShapeDtypeStruct((NPAD, D), _F32),
                   jax.ShapeDtypeStruct((NPAD, D), _F32),
                   jax.ShapeDtypeStruct((NPAD, DW), _F32),
                   jax.ShapeDtypeStruct((NPAD, DW), _F32)],
    )(x_p, dout, din)


def _combine_h(su_ref, st_ref, io_ref, ii_ref, ws_ref, bs_ref, wd_ref, bd_ref):
    io = io_ref[:, 0:1]
    ii = ii_ref[:, 0:1]
    a = io * su_ref[...]
    b = ii * st_ref[...]
    h = 0.5 * (jnp.dot(a, ws_ref[...], precision=_HI, preferred_element_type=_F32)
               + bs_ref[0:1, :])
    h += 0.5 * (jnp.dot(b, wd_ref[...], precision=_HI, preferred_element_type=_F32)
                + bd_ref[0:1, :])
    return jnp.maximum(h, 0.0)


def _combine_body(su_ref, st_ref, io_ref, ii_ref, ws_ref, bs_ref, wd_ref, bd_ref,
                  u_ref, v_ref):
    h = _combine_h(su_ref, st_ref, io_ref, ii_ref, ws_ref, bs_ref, wd_ref, bd_ref)
    u_ref[...] = ii_ref[:, 0:1] * h
    v_ref[...] = io_ref[:, 0:1] * h


def _tc_combine(su, stv, io16, ii16, Ws, bsb, Wd, bdb):
    bs_d = pl.BlockSpec((BLK, D), lambda i: (i, 0))
    bs_w = pl.BlockSpec((BLK, DW), lambda i: (i, 0))
    bs_m = pl.BlockSpec((D, D), lambda i: (0, 0))
    bs_b = pl.BlockSpec((8, D), lambda i: (0, 0))
    return pl.pallas_call(
        _combine_body,
        grid=(NB,),
        in_specs=[bs_d, bs_d, bs_w, bs_w, bs_m, bs_b, bs_m, bs_b],
        out_specs=[bs_d, bs_d],
        out_shape=[jax.ShapeDtypeStruct((NPAD, D), _F32),
                   jax.ShapeDtypeStruct((NPAD, D), _F32)],
    )(su, stv, io16, ii16, Ws, bsb, Wd, bdb)


def _final_body(su_ref, st_ref, io_ref, ii_ref, ws_ref, bs_ref, wd_ref, bd_ref,
                batch_ref, wl1_ref, bl1_ref, wl2_ref, bl2_ref, out_ref, pool_scr):
    i = pl.program_id(0)

    @pl.when(i == 0)
    def _():
        pool_scr[...] = jnp.full((G, D), -jnp.inf, _F32)

    h = _combine_h(su_ref, st_ref, io_ref, ii_ref, ws_ref, bs_ref, wd_ref, bd_ref)
    b = batch_ref[:, 0:1]

    for g in range(G):
        m = (b == g)
        red = jnp.max(jnp.where(m, h, -jnp.inf), axis=0, keepdims=True)
        pool_scr[g:g + 1, :] = jnp.maximum(pool_scr[g:g + 1, :], red)

    @pl.when(i == NB - 1)
    def _():
        p = pool_scr[...]
        o1 = jnp.maximum(
            jnp.dot(p, wl1_ref[...], precision=_HI, preferred_element_type=_F32)
            + bl1_ref[0:1, :], 0.0)
        out_ref[...] = (jnp.dot(o1, wl2_ref[...], precision=_HI,
                                preferred_element_type=_F32) + bl2_ref[0:1, :])


def _tc_final(su, stv, io16, ii16, Ws, bsb, Wd, bdb, batch3, Wl1p, bl1b, Wl2p, bl2b):
    bs_d = pl.BlockSpec((BLK, D), lambda i: (i, 0))
    bs_w = pl.BlockSpec((BLK, DW), lambda i: (i, 0))
    bs_m = pl.BlockSpec((D, D), lambda i: (0, 0))
    bs_b = pl.BlockSpec((8, D), lambda i: (0, 0))
    bs_i = pl.BlockSpec((BLK, DW), lambda i: (i, 0))
    return pl.pallas_call(
        _final_body,
        grid=(NB,),
        in_specs=[bs_d, bs_d, bs_w, bs_w, bs_m, bs_b, bs_m, bs_b,
                  bs_i, bs_m, bs_b, bs_m, bs_b],
        out_specs=pl.BlockSpec((G, D), lambda i: (0, 0)),
        out_shape=jax.ShapeDtypeStruct((G, D), _F32),
        scratch_shapes=[pltpu.VMEM((G, D), _F32)],
    )(su, stv, io16, ii16, Ws, bsb, Wd, bdb, batch3, Wl1p, bl1b, Wl2p, bl2b)


def kernel(x, edge_index, batch, W1s, b1s, W1d, b1d, W2s, b2s, W2d, b2d,
           W3s, b3s, W3d, b3d, Wl1, bl1, Wl2, bl2):
    row, col = edge_index[0], edge_index[1]

    pad = jnp.full((NT, EPTP - EPT), NPAD - 1, jnp.int32)
    rowp = jnp.concatenate([row.reshape(NT, EPT), pad], axis=1).reshape(NT, CH, K)
    colp = jnp.concatenate([col.reshape(NT, EPT), pad], axis=1).reshape(NT, CH, K)

    x_p = jnp.pad(x, ((0, NPAD - N), (0, 0)))
    batch3 = jnp.broadcast_to(
        jnp.pad(batch, (0, NPAD - N), constant_values=G)[:, None], (NPAD, DW))

    bcast8 = lambda vec: jnp.broadcast_to(
        jnp.pad(vec, (0, D - vec.shape[0]))[None, :], (8, D))
    b1sb, b1db = bcast8(b1s), bcast8(b1d)
    b2sb, b2db = bcast8(b2s), bcast8(b2d)
    b3sb, b3db = bcast8(b3s), bcast8(b3d)
    Wl1p = jnp.zeros((D, D), _F32).at[:, :Wl1.shape[1]].set(Wl1)
    Wl2p = jnp.zeros((D, D), _F32).at[:Wl2.shape[0], :Wl2.shape[1]].set(Wl2)
    bl1b, bl2b = bcast8(bl1), bcast8(bl2)

    dout, din = _sc_degrees(rowp, colp)
    u, v, io16, ii16 = _tc_prep(x_p, dout, din)

    su, stv = _sc_spmm(u, v, rowp, colp)
    u, v = _tc_combine(su, stv, io16, ii16, W1s, b1sb, W1d, b1db)

    su, stv = _sc_spmm(u, v, rowp, colp)
    u, v = _tc_combine(su, stv, io16, ii16, W2s, b2sb, W2d, b2db)

    su, stv = _sc_spmm(u, v, rowp, colp)
    out128 = _tc_final(su, stv, io16, ii16, W3s, b3sb, W3d, b3db,
                       batch3, Wl1p, bl1b, Wl2p, bl2b)

    return out128[:, :1]
